# baseline (speedup 1.0000x reference)
import functools

import jax
import jax.numpy as jnp
from jax import lax
from jax.experimental import pallas as pl
from jax.experimental.pallas import tpu as pltpu
from jax.experimental.pallas import tpu_sc as plsc

BATCH = 4096
SEQ = 77
SEQP = 80
EMBED = 768
LANES = 16
NCOL = EMBED // LANES
U = 24
NU = 3
TAILR = SEQ - NU * U
MINI = 8


def kernel(input_ids, token_table, pos_table):
    info = plsc.get_sparse_core_info()
    nw = info.num_cores * info.num_subcores
    seq_per_w = BATCH // nw

    mesh = plsc.VectorSubcoreMesh(core_axis_name="c", subcore_axis_name="s")

    @functools.partial(
        pl.kernel,
        out_type=jax.ShapeDtypeStruct((BATCH, SEQ, EMBED), jnp.float32),
        mesh=mesh,
        scratch_types=[
            pltpu.VMEM((2 * SEQP,), jnp.int32),
            pltpu.VMEM((SEQ * EMBED,), jnp.float32),
            pltpu.VMEM((1, U, EMBED), jnp.float32),
            pltpu.VMEM((1, U, EMBED), jnp.float32),
            pltpu.VMEM((1, U, EMBED), jnp.float32),
            pltpu.VMEM((MINI, EMBED), jnp.float32),
            pltpu.VMEM((1, TAILR, EMBED), jnp.float32),
            pltpu.SemaphoreType.DMA,
            pltpu.SemaphoreType.DMA,
            pltpu.SemaphoreType.DMA,
            pltpu.SemaphoreType.DMA,
            pltpu.SemaphoreType.DMA,
            pltpu.SemaphoreType.DMA,
        ],
    )
    def run(ids_hbm, tok_hbm, pos_hbm, out_hbm, ids_v, pos_v, b0, b1, b2,
            mini_v, tail_v, g0, g1, g2, gm, ts, ids_sem):
        wid = lax.axis_index("s") * info.num_cores + lax.axis_index("c")
        base = wid * seq_per_w
        bufs = (b0, b1, b2)
        gsems = (g0, g1, g2)
        pltpu.sync_copy(pos_hbm, pos_v)

        def idx(i, u):
            off = lax.rem(i, 2) * SEQP + u * U
            return ids_v.at[pl.ds(off, U)]

        def idx_mini(i):
            off = lax.rem(i, 2) * SEQP + NU * U
            return ids_v.at[pl.ds(off, MINI)]

        def fire_ids(i):
            pltpu.async_copy(
                ids_hbm.at[pl.ds((base + i) * SEQP, SEQP)],
                ids_v.at[pl.ds(lax.rem(i, 2) * SEQP, SEQP)], ids_sem)

        pltpu.sync_copy(ids_hbm.at[pl.ds(base * SEQP, SEQP)],
                        ids_v.at[pl.ds(0, SEQP)])
        fire_ids(1)
        for u in range(NU):
            pltpu.async_copy(tok_hbm.at[idx(0, u)], bufs[u].at[0], gsems[u])
        pltpu.async_copy(tok_hbm.at[idx_mini(0)], mini_v, gm)

        def seq_body(i, carry):
            seq = base + i

            def add_unit(u):
                pltpu.make_async_copy(tok_hbm.at[idx(i, u)], bufs[u].at[0],
                                      gsems[u]).wait()

                def row_body(r, c3):
                    pbase = (u * U + r) * EMBED
                    for c in range(NCOL):
                        sl = pl.ds(c * LANES, LANES)
                        bufs[u][0, r, sl] = (
                            bufs[u][0, r, sl]
                            + pos_v[pl.ds(pbase + c * LANES, LANES)])
                    return c3

                lax.fori_loop(0, U, row_body, 0)
                return pltpu.async_copy(
                    bufs[u], out_hbm.at[pl.ds(seq, 1), pl.ds(u * U, U)],
                    gsems[u])

            st0 = add_unit(0)
            st1 = add_unit(1)

            @pl.when(i < seq_per_w - 1)
            def _():
                pltpu.make_async_copy(
                    ids_hbm.at[pl.ds(base * SEQP, SEQP)],
                    ids_v.at[pl.ds(0, SEQP)], ids_sem).wait()

            st0.wait()

            @pl.when(i < seq_per_w - 1)
            def _():
                pltpu.async_copy(tok_hbm.at[idx(i + 1, 0)], bufs[0].at[0],
                                 gsems[0])

            st2 = add_unit(2)
            st1.wait()

            @pl.when(i < seq_per_w - 1)
            def _():
                pltpu.async_copy(tok_hbm.at[idx(i + 1, 1)], bufs[1].at[0],
                                 gsems[1])

            @pl.when(i > 0)
            def _():
                pltpu.make_async_copy(
                    tail_v, out_hbm.at[pl.ds(seq, 1), pl.ds(NU * U, TAILR)],
                    ts).wait()

            pltpu.make_async_copy(tok_hbm.at[idx_mini(i)], mini_v, gm).wait()

            def tail_body(t, c3):
                pbase = (NU * U + t) * EMBED
                for c in range(NCOL):
                    sl = pl.ds(c * LANES, LANES)
                    tail_v[0, t, sl] = (
                        mini_v[t, sl]
                        + pos_v[pl.ds(pbase + c * LANES, LANES)])
                return c3

            lax.fori_loop(0, TAILR, tail_body, 0)
            pltpu.async_copy(tail_v,
                             out_hbm.at[pl.ds(seq, 1), pl.ds(NU * U, TAILR)],
                             ts)
            st2.wait()

            @pl.when(i < seq_per_w - 1)
            def _():
                pltpu.async_copy(tok_hbm.at[idx(i + 1, 2)], bufs[2].at[0],
                                 gsems[2])
                pltpu.async_copy(tok_hbm.at[idx_mini(i + 1)], mini_v, gm)

            @pl.when(i < seq_per_w - 2)
            def _():
                fire_ids(i + 2)

            return carry

        lax.fori_loop(0, seq_per_w, seq_body, 0)
        pltpu.make_async_copy(
            tail_v,
            out_hbm.at[pl.ds(base + seq_per_w - 1, 1), pl.ds(NU * U, TAILR)],
            ts).wait()

    ids_pad = jnp.pad(input_ids.astype(jnp.int32), ((0, 0), (0, SEQP - SEQ)))
    return run(ids_pad.reshape(-1), token_table, pos_table.reshape(-1))
